# baseline (speedup 1.0000x reference)
import functools

import jax
import jax.numpy as jnp
from jax.experimental import pallas as pl
from jax.experimental.pallas import tpu as pltpu
from jax.experimental.pallas import tpu_sc as plsc

_NUMBERS = (99991, 100003, 100019)
_D = 64
_W = 128


def _mod_body(id_ref, start_ref, r0_ref, r1_ref, r2_ref):
    idx = id_ref[...] + start_ref[...]
    r0_ref[...] = idx % _NUMBERS[0]
    r1_ref[...] = idx % _NUMBERS[1]
    r2_ref[...] = idx % _NUMBERS[2]


def _mod_hashes(id, start):
    B, L = id.shape
    RB = 512
    out = jax.ShapeDtypeStruct((B, L), jnp.int32)
    return pl.pallas_call(
        _mod_body,
        grid=(B // RB,),
        in_specs=[
            pl.BlockSpec((RB, L), lambda i: (i, 0)),
            pl.BlockSpec((RB, 1), lambda i: (i, 0)),
        ],
        out_specs=[
            pl.BlockSpec((RB, L), lambda i: (i, 0)),
            pl.BlockSpec((RB, L), lambda i: (i, 0)),
            pl.BlockSpec((RB, L), lambda i: (i, 0)),
        ],
        out_shape=[out, out, out],
    )(id, start)


_NW = 32


def _sc_gather_sum(e0, e1, e2, r0, r1, r2):
    nrow = r0.shape[0]
    ntok = nrow * 128
    SP = nrow // _NW
    mesh = plsc.VectorSubcoreMesh(core_axis_name="c", subcore_axis_name="s")

    @functools.partial(
        pl.kernel,
        out_type=jax.ShapeDtypeStruct((ntok // 2, 2 * _D), jnp.float32),
        mesh=mesh,
        compiler_params=pltpu.CompilerParams(use_tc_tiling_on_sc=False),
        scratch_types=(
            [pltpu.VMEM((128, _D), jnp.float32)] * 9
            + [pltpu.VMEM((128,), jnp.int32)] * 9
            + [pltpu.SMEM((1,), jnp.int32)]
            + [pltpu.SemaphoreType.DMA] * 9
        ),
    )
    def k(e0_hbm, e1_hbm, e2_hbm, r0_hbm, r1_hbm, r2_hbm, o_hbm,
          ga0, ga1, ga2, gb0, gb1, gb2, gc0, gc1, gc2,
          ia0, ia1, ia2, ib0, ib1, ib2, ic0, ic1, ic2, cnt,
          sa0, sa1, sa2, sb0, sb1, sb2, sc0, sc1, sc2):
        cnt[0] = 0
        bufs = ((ga0, ga1, ga2), (gb0, gb1, gb2), (gc0, gc1, gc2))
        ibufs = ((ia0, ia1, ia2), (ib0, ib1, ib2), (ic0, ic1, ic2))
        sems = ((sa0, sa1, sa2), (sb0, sb1, sb2), (sc0, sc1, sc2))

        def _sum_into(o_v, g0, g1, g2):
            @pl.loop(0, 64)
            def _(p):
                for t in range(2):
                    i = 2 * p + t
                    for j in range(0, _D, 16):
                        src = (i, pl.ds(j, 16))
                        o_v[p, pl.ds(t * _D + j, 16)] = g0[src] + g1[src] + g2[src]

        def body(r0_v, r1_v, r2_v, o_v):
            s = cnt[0]
            par = jax.lax.rem(s, 3)

            for b in range(3):
                @pl.when(jnp.logical_and(s < SP, par == b))
                def _(b=b):
                    for t, r_v in enumerate((r0_v, r1_v, r2_v)):
                        for c in range(0, 128, 16):
                            ibufs[b][t][pl.ds(c, 16)] = r_v[0, pl.ds(c, 16)]
                    pltpu.async_copy(e0_hbm.at[ibufs[b][0]], bufs[b][0], sems[b][0])
                    pltpu.async_copy(e1_hbm.at[ibufs[b][1]], bufs[b][1], sems[b][1])
                    pltpu.async_copy(e2_hbm.at[ibufs[b][2]], bufs[b][2], sems[b][2])

            for b in range(3):
                @pl.when(jnp.logical_and(s > 1, par == (b + 2) % 3))
                def _(b=b):
                    pltpu.make_async_copy(e0_hbm.at[pl.ds(0, 128)], bufs[b][0], sems[b][0]).wait()
                    pltpu.make_async_copy(e1_hbm.at[pl.ds(0, 128)], bufs[b][1], sems[b][1]).wait()
                    pltpu.make_async_copy(e2_hbm.at[pl.ds(0, 128)], bufs[b][2], sems[b][2]).wait()
                    _sum_into(o_v, *bufs[b])

            cnt[0] = s + 1

        pltpu.emit_pipeline(
            body,
            grid=(_NW, SP + 2),
            in_specs=[
                pl.BlockSpec((1, 128), lambda w, s: (w * SP + jnp.minimum(s, SP - 1), 0)),
                pl.BlockSpec((1, 128), lambda w, s: (w * SP + jnp.minimum(s, SP - 1), 0)),
                pl.BlockSpec((1, 128), lambda w, s: (w * SP + jnp.minimum(s, SP - 1), 0)),
            ],
            out_specs=[
                pl.BlockSpec((64, 2 * _D), lambda w, s: (w * SP + jnp.maximum(s - 2, 0), 0)),
            ],
            core_axis_name=("c", "s"),
            dimension_semantics=(pltpu.PARALLEL, pltpu.ARBITRARY),
        )(r0_hbm, r1_hbm, r2_hbm, o_hbm)

    return k(e0, e1, e2, r0, r1, r2)


def _ln_body(pe_ref, g_ref, b_ref, o_ref):
    x = pe_ref[...]
    lane = jax.lax.broadcasted_iota(jnp.int32, x.shape, 1)
    left = lane < _D
    xl = jnp.where(left, x, 0.0)
    xx = x * x
    s_all = jnp.sum(x, axis=-1, keepdims=True)
    s_l = jnp.sum(xl, axis=-1, keepdims=True)
    q_all = jnp.sum(xx, axis=-1, keepdims=True)
    q_l = jnp.sum(jnp.where(left, xx, 0.0), axis=-1, keepdims=True)
    mu_l = s_l / _D
    mu_r = (s_all - s_l) / _D
    rs_l = jax.lax.rsqrt(q_l / _D - mu_l * mu_l + 1e-5)
    rs_r = jax.lax.rsqrt((q_all - q_l) / _D - mu_r * mu_r + 1e-5)
    mu = jnp.where(left, mu_l, mu_r)
    rs = jnp.where(left, rs_l, rs_r)
    o_ref[...] = (x - mu) * rs * g_ref[...] + b_ref[...]


def _layer_norm(pe, gamma, beta):
    nrow = pe.shape[0]
    TB = 2048
    return pl.pallas_call(
        _ln_body,
        grid=(nrow // TB,),
        in_specs=[
            pl.BlockSpec((TB, 2 * _D), lambda i: (i, 0)),
            pl.BlockSpec((1, 2 * _D), lambda i: (0, 0)),
            pl.BlockSpec((1, 2 * _D), lambda i: (0, 0)),
        ],
        out_specs=pl.BlockSpec((TB, 2 * _D), lambda i: (i, 0)),
        out_shape=jax.ShapeDtypeStruct((nrow, 2 * _D), jnp.float32),
    )(pe, gamma, beta)


def kernel(id, start, emb0, emb1, emb2, gamma, beta):
    B, L = id.shape
    ntok = B * L
    r0, r1, r2 = _mod_hashes(id, start)
    nr = ntok // 128
    pe = _sc_gather_sum(
        emb0, emb1, emb2,
        r0.reshape(nr, 128), r1.reshape(nr, 128), r2.reshape(nr, 128),
    )
    g2 = jnp.concatenate([gamma, gamma]).reshape(1, 2 * _D)
    b2 = jnp.concatenate([beta, beta]).reshape(1, 2 * _D)
    out = _layer_norm(pe, g2, b2)
    return out.reshape(B, L, _D)
